# baseline (speedup 1.0000x reference)
import functools

import jax
import jax.numpy as jnp
from jax import lax
from jax.experimental import pallas as pl
from jax.experimental.pallas import tpu as pltpu
from jax.experimental.pallas import tpu_sc as plsc

B = 8
P = 1024
K = 20
N = B * P

_f32 = jnp.float32

_NC = 2
_NS = 16
_NW = _NC * _NS
_SUB = 16


def _dot(a, b):
    return jax.lax.dot_general(a, b, (((1,), (0,)), ((), ())),
                               preferred_element_type=_f32)


def _dot00(a, b):
    return jax.lax.dot_general(a, b, (((0,), (0,)), ((), ())),
                               preferred_element_type=_f32)


def _topk_idx(d2, k):
    iota0 = jax.lax.broadcasted_iota(jnp.int32, (P, P), 0)
    rows = []
    for _ in range(k):
        m = jnp.min(d2, axis=0, keepdims=True)
        eq = d2 == m
        am = jnp.min(jnp.where(eq, iota0, P * 2), axis=0,
                     keepdims=True)
        rows.append(am)
        d2 = jnp.where(eq, _f32(1e30), d2)
    return jnp.concatenate(rows, axis=0)


def _pairwise_d2(xt):
    xx = _dot00(xt, xt)
    sq = xt * xt
    sqc = _dot00(sq, jnp.ones((xt.shape[0], 1), _f32))
    sqr = _dot(jnp.ones((1, xt.shape[0]), _f32), sq)
    d2 = sqc + sqr - 2.0 * xx
    iota0 = jax.lax.broadcasted_iota(jnp.int32, (P, P), 0)
    iota1 = jax.lax.broadcasted_iota(jnp.int32, (P, P), 1)
    return d2 + jnp.where(iota0 == iota1, _f32(1e10), _f32(0.0))


def _eye(n):
    return (jax.lax.broadcasted_iota(jnp.int32, (n, n), 0)
            == jax.lax.broadcasted_iota(jnp.int32, (n, n), 1)).astype(_f32)


def _stage1_kernel(xt_ref, wu1_ref, wv1_ref, b1a_ref, w1b_ref, b1b_ref,
                   w1c_ref, b1c_ref, w2u_ref, w2v_ref, b2_ref,
                   x1_ref, u2_ref, v2_ref, idx_ref):
    b = pl.program_id(0)
    xt = xt_ref[0]
    iota0 = jax.lax.broadcasted_iota(jnp.int32, (P, P), 0)

    idx1 = _topk_idx(_pairwise_d2(xt), K)
    u1 = _dot(wu1_ref[...], xt) + b1a_ref[...]
    v1 = _dot(wv1_ref[...], xt)
    acc1 = None
    for k in range(K):
        oh = (iota0 == idx1[k:k + 1, :]).astype(_f32)
        h = jnp.maximum(u1 + _dot(v1, oh), 0.0)
        h = jnp.maximum(_dot(w1b_ref[...], h) + b1b_ref[...], 0.0)
        h = _dot(w1c_ref[...], h) + b1c_ref[...]
        acc1 = h if acc1 is None else jnp.maximum(acc1, h)
    x1t = acc1

    idx2t = _topk_idx(_pairwise_d2(x1t), K)
    x1_ref[0] = _dot00(x1t, _eye(64))
    u2_ref[0] = _dot00(x1t, w2u_ref[...]) + b2_ref[...]
    v2_ref[0] = _dot00(x1t, w2v_ref[...])
    idx_rows = _dot00(idx2t.astype(_f32), _eye(K))
    idx_ref[0] = idx_rows.astype(jnp.int32) + b * P


@functools.lru_cache(maxsize=2)
def _get_sc_gather_max(npts):
    pts_w = npts // _NW

    @functools.partial(
        pl.kernel,
        out_type=jax.ShapeDtypeStruct((npts, 128), _f32),
        mesh=plsc.VectorSubcoreMesh(core_axis_name="c", subcore_axis_name="s",
                                    num_cores=_NC, num_subcores=_NS),
        scratch_types=[
            pltpu.VMEM((_SUB * K,), jnp.int32),
            pltpu.VMEM((_SUB * K,), jnp.int32),
            pltpu.VMEM((_SUB * K, 128), _f32),
            pltpu.VMEM((_SUB * K, 128), _f32),
            pltpu.VMEM((_SUB, 128), _f32),
            pltpu.SemaphoreType.DMA,
            pltpu.SemaphoreType.DMA,
        ],
    )
    def _sc_gather_max(v2_hbm, idx_hbm, out_hbm, idx0_v, idx1_v, rows0_v,
                       rows1_v, acc_v, sem0, sem1):
        wid = lax.axis_index("s") * _NC + lax.axis_index("c")
        idx_bufs = (idx0_v, idx1_v)
        row_bufs = (rows0_v, rows1_v)
        sems = (sem0, sem1)

        def start(sc):
            pbase = wid * pts_w + sc * _SUB
            s = sc % 2
            pltpu.sync_copy(idx_hbm.at[pl.ds(pbase * K, _SUB * K)],
                            idx_bufs[s])
            pltpu.async_copy(v2_hbm.at[idx_bufs[s]], row_bufs[s], sems[s])

        start(0)
        nsub = pts_w // _SUB
        for sc in range(nsub):
            if sc + 1 < nsub:
                start(sc + 1)
            s = sc % 2
            rows_v = row_bufs[s]
            pltpu.make_async_copy(v2_hbm.at[idx_bufs[s]], rows_v,
                                  sems[s]).wait()

            def body(p, carry):
                for f in range(8):
                    a = rows_v[p * K, pl.ds(f * 16, 16)]
                    for k in range(1, K):
                        a = jnp.maximum(a, rows_v[p * K + k,
                                                  pl.ds(f * 16, 16)])
                    acc_v[p, pl.ds(f * 16, 16)] = a
                return carry

            lax.fori_loop(0, _SUB, body, 0)
            pbase = wid * pts_w + sc * _SUB
            pltpu.sync_copy(acc_v, out_hbm.at[pl.ds(pbase, _SUB)])

    return _sc_gather_max


def _stage2_kernel(x1_ref, u2_ref, m_ref, wl_ref, bl_ref, pool_ref):
    x2 = u2_ref[0] + m_ref[...]
    feat = jnp.concatenate([x1_ref[0], x2], axis=1)
    out = _dot(feat, wl_ref[...]) + bl_ref[...]
    pool_ref[0] = jnp.max(out, axis=0, keepdims=True)


def _head_kernel(p_ref, w1_ref, b1_ref, w2_ref, b2_ref, w3_ref, b3_ref,
                 out_ref):
    h = jnp.maximum(_dot(p_ref[...], w1_ref[...]) + b1_ref[...], 0.0)
    h = jnp.maximum(_dot(h, w2_ref[...]) + b2_ref[...], 0.0)
    logit = _dot(h, w3_ref[...]) + b3_ref[...]
    mx = jnp.max(logit, axis=1, keepdims=True)
    s = logit - mx
    out_ref[...] = s - jnp.log(jnp.sum(jnp.exp(s), axis=1, keepdims=True))


def _full(shape):
    nd = len(shape)
    return pl.BlockSpec(shape, lambda *_: (0,) * nd)


@jax.jit
def kernel(data, batch, W1a, b1a, W1b, b1b, W1c, b1c, W2, b2, Wl, bl,
           Wm1, bm1, Wm2, bm2, Wm3, bm3):
    del batch
    xt = data.reshape(B, P, 2).transpose(0, 2, 1)
    wu1 = (W1a[:2] - W1a[2:]).T
    wv1 = W1a[2:].T
    w2u = W2[:64] - W2[64:]
    w2v = W2[64:]

    col = lambda v: v[:, None]
    row = lambda v: v[None, :]

    BH = B // 2
    NH = BH * P
    pooled_halves = []
    for h in range(2):
        x1r, u2r, v2r, idx2 = pl.pallas_call(
            _stage1_kernel,
            grid=(BH,),
            in_specs=[
                pl.BlockSpec((1, 2, P), lambda b: (b, 0, 0)),
                _full((64, 2)), _full((64, 2)), _full((64, 1)),
                _full((64, 64)), _full((64, 1)),
                _full((64, 64)), _full((64, 1)),
                _full((64, 128)), _full((64, 128)), _full((1, 128)),
            ],
            out_specs=[
                pl.BlockSpec((1, P, 64), lambda b: (b, 0, 0)),
                pl.BlockSpec((1, P, 128), lambda b: (b, 0, 0)),
                pl.BlockSpec((1, P, 128), lambda b: (b, 0, 0)),
                pl.BlockSpec((1, P, K), lambda b: (b, 0, 0)),
            ],
            out_shape=[
                jax.ShapeDtypeStruct((BH, P, 64), _f32),
                jax.ShapeDtypeStruct((BH, P, 128), _f32),
                jax.ShapeDtypeStruct((BH, P, 128), _f32),
                jax.ShapeDtypeStruct((BH, P, K), jnp.int32),
            ],
        )(xt[h * BH:(h + 1) * BH], wu1, wv1, col(b1a), W1b.T, col(b1b),
          W1c.T, col(b1c), w2u, w2v, row(b2))

        m = _get_sc_gather_max(NH)(v2r.reshape(NH, 128),
                                   idx2.reshape(NH * K))

        pooled_halves.append(pl.pallas_call(
            _stage2_kernel,
            grid=(BH,),
            in_specs=[
                pl.BlockSpec((1, P, 64), lambda b: (b, 0, 0)),
                pl.BlockSpec((1, P, 128), lambda b: (b, 0, 0)),
                pl.BlockSpec((P, 128), lambda b: (b, 0)),
                _full((192, 1024)), _full((1, 1024)),
            ],
            out_specs=pl.BlockSpec((1, 1, 1024), lambda b: (b, 0, 0)),
            out_shape=jax.ShapeDtypeStruct((BH, 1, 1024), _f32),
        )(x1r, u2r, m, Wl, row(bl)))

    pooled = jnp.concatenate(pooled_halves, axis=0)

    return pl.pallas_call(
        _head_kernel,
        in_specs=[_full((B, 1024)),
                  _full((1024, 512)), _full((1, 512)),
                  _full((512, 256)), _full((1, 256)),
                  _full((256, 16)), _full((1, 16))],
        out_specs=_full((B, 16)),
        out_shape=jax.ShapeDtypeStruct((B, 16), _f32),
    )(pooled.reshape(B, 1024), Wm1, row(bm1), Wm2, row(bm2), Wm3, row(bm3))


# speedup vs baseline: 1.0362x; 1.0362x over previous
import functools

import jax
import jax.numpy as jnp
from jax import lax
from jax.experimental import pallas as pl
from jax.experimental.pallas import tpu as pltpu
from jax.experimental.pallas import tpu_sc as plsc

B = 8
P = 1024
K = 20
N = B * P

_f32 = jnp.float32

_NC = 2
_NS = 16
_NW = _NC * _NS
_PTS_W = N // _NW
_SUB = 16
_NSUB = _PTS_W // _SUB


def _dot(a, b):
    return jax.lax.dot_general(a, b, (((1,), (0,)), ((), ())),
                               preferred_element_type=_f32)


def _dot00(a, b):
    return jax.lax.dot_general(a, b, (((0,), (0,)), ((), ())),
                               preferred_element_type=_f32)


def _topk_idx(d2, k):
    iota0 = jax.lax.broadcasted_iota(jnp.int32, (P, P), 0)
    rows = []
    for _ in range(k):
        m = jnp.min(d2, axis=0, keepdims=True)
        eq = d2 == m
        am = jnp.min(jnp.where(eq, iota0, P * 2), axis=0,
                     keepdims=True)
        rows.append(am)
        d2 = jnp.where(eq, _f32(1e30), d2)
    return jnp.concatenate(rows, axis=0)


def _pairwise_d2(xt):
    xx = _dot00(xt, xt)
    sq = xt * xt
    sqc = _dot00(sq, jnp.ones((xt.shape[0], 1), _f32))
    sqr = _dot(jnp.ones((1, xt.shape[0]), _f32), sq)
    d2 = sqc + sqr - 2.0 * xx
    iota0 = jax.lax.broadcasted_iota(jnp.int32, (P, P), 0)
    iota1 = jax.lax.broadcasted_iota(jnp.int32, (P, P), 1)
    return d2 + jnp.where(iota0 == iota1, _f32(1e10), _f32(0.0))


def _eye(n):
    return (jax.lax.broadcasted_iota(jnp.int32, (n, n), 0)
            == jax.lax.broadcasted_iota(jnp.int32, (n, n), 1)).astype(_f32)


def _stage1_kernel(xt_ref, wu1_ref, wv1_ref, b1a_ref, w1b_ref, b1b_ref,
                   w1c_ref, b1c_ref, w2u_ref, w2v_ref, b2_ref,
                   x1_ref, u2_ref, v2_ref, idx_ref):
    b = pl.program_id(0)
    xt = xt_ref[0]
    iota0 = jax.lax.broadcasted_iota(jnp.int32, (P, P), 0)

    idx1 = _topk_idx(_pairwise_d2(xt), K)
    u1 = _dot(wu1_ref[...], xt) + b1a_ref[...]
    v1 = _dot(wv1_ref[...], xt)
    acc1 = None
    for k in range(K):
        oh = (iota0 == idx1[k:k + 1, :]).astype(_f32)
        h = jnp.maximum(u1 + _dot(v1, oh), 0.0)
        h = jnp.maximum(_dot(w1b_ref[...], h) + b1b_ref[...], 0.0)
        h = _dot(w1c_ref[...], h) + b1c_ref[...]
        acc1 = h if acc1 is None else jnp.maximum(acc1, h)
    x1t = acc1

    idx2t = _topk_idx(_pairwise_d2(x1t), K)
    x1_ref[0] = _dot00(x1t, _eye(64))
    u2_ref[0] = _dot00(x1t, w2u_ref[...]) + b2_ref[...]
    v2_ref[0] = _dot00(x1t, w2v_ref[...])
    idx_rows = _dot00(idx2t.astype(_f32), _eye(K))
    idx_ref[0] = idx_rows.astype(jnp.int32) + b * P


@functools.lru_cache(maxsize=1)
def _get_sc_gather_max():
    @functools.partial(
        pl.kernel,
        out_type=jax.ShapeDtypeStruct((N, 128), _f32),
        mesh=plsc.VectorSubcoreMesh(core_axis_name="c", subcore_axis_name="s",
                                    num_cores=_NC, num_subcores=_NS),
        scratch_types=[
            pltpu.VMEM((_SUB * K,), jnp.int32),
            pltpu.VMEM((_SUB * K,), jnp.int32),
            pltpu.VMEM((_SUB * K, 128), _f32),
            pltpu.VMEM((_SUB * K, 128), _f32),
            pltpu.VMEM((_SUB, 128), _f32),
            pltpu.SemaphoreType.DMA,
            pltpu.SemaphoreType.DMA,
        ],
    )
    def _sc_gather_max(v2_hbm, idx_hbm, out_hbm, idx0_v, idx1_v, rows0_v,
                       rows1_v, acc_v, sem0, sem1):
        wid = lax.axis_index("s") * _NC + lax.axis_index("c")
        idx_bufs = (idx0_v, idx1_v)
        row_bufs = (rows0_v, rows1_v)
        sems = (sem0, sem1)

        def start(sc):
            pbase = wid * _PTS_W + sc * _SUB
            s = sc % 2
            pltpu.sync_copy(idx_hbm.at[pl.ds(pbase * K, _SUB * K)],
                            idx_bufs[s])
            pltpu.async_copy(v2_hbm.at[idx_bufs[s]], row_bufs[s], sems[s])

        start(0)
        for sc in range(_NSUB):
            if sc + 1 < _NSUB:
                start(sc + 1)
            s = sc % 2
            rows_v = row_bufs[s]
            pltpu.make_async_copy(v2_hbm.at[idx_bufs[s]], rows_v,
                                  sems[s]).wait()

            def body(p, carry):
                for f in range(8):
                    a = rows_v[p * K, pl.ds(f * 16, 16)]
                    for k in range(1, K):
                        a = jnp.maximum(a, rows_v[p * K + k,
                                                  pl.ds(f * 16, 16)])
                    acc_v[p, pl.ds(f * 16, 16)] = a
                return carry

            lax.fori_loop(0, _SUB, body, 0)
            pbase = wid * _PTS_W + sc * _SUB
            pltpu.sync_copy(acc_v, out_hbm.at[pl.ds(pbase, _SUB)])

    return _sc_gather_max


def _stage2_kernel(x1_ref, u2_ref, m_ref, wl_ref, bl_ref, pool_ref):
    x2 = u2_ref[0] + m_ref[...]
    feat = jnp.concatenate([x1_ref[0], x2], axis=1)
    out = _dot(feat, wl_ref[...]) + bl_ref[...]
    pool_ref[0] = jnp.max(out, axis=0, keepdims=True)


def _head_kernel(p_ref, w1_ref, b1_ref, w2_ref, b2_ref, w3_ref, b3_ref,
                 out_ref):
    h = jnp.maximum(_dot(p_ref[...], w1_ref[...]) + b1_ref[...], 0.0)
    h = jnp.maximum(_dot(h, w2_ref[...]) + b2_ref[...], 0.0)
    logit = _dot(h, w3_ref[...]) + b3_ref[...]
    mx = jnp.max(logit, axis=1, keepdims=True)
    s = logit - mx
    out_ref[...] = s - jnp.log(jnp.sum(jnp.exp(s), axis=1, keepdims=True))


def _full(shape):
    nd = len(shape)
    return pl.BlockSpec(shape, lambda *_: (0,) * nd)


@jax.jit
def kernel(data, batch, W1a, b1a, W1b, b1b, W1c, b1c, W2, b2, Wl, bl,
           Wm1, bm1, Wm2, bm2, Wm3, bm3):
    del batch
    xt = data.reshape(B, P, 2).transpose(0, 2, 1)
    wu1 = (W1a[:2] - W1a[2:]).T
    wv1 = W1a[2:].T
    w2u = W2[:64] - W2[64:]
    w2v = W2[64:]

    col = lambda v: v[:, None]
    row = lambda v: v[None, :]

    x1r, u2r, v2r, idx2 = pl.pallas_call(
        _stage1_kernel,
        grid=(B,),
        in_specs=[
            pl.BlockSpec((1, 2, P), lambda b: (b, 0, 0)),
            _full((64, 2)), _full((64, 2)), _full((64, 1)),
            _full((64, 64)), _full((64, 1)),
            _full((64, 64)), _full((64, 1)),
            _full((64, 128)), _full((64, 128)), _full((1, 128)),
        ],
        out_specs=[
            pl.BlockSpec((1, P, 64), lambda b: (b, 0, 0)),
            pl.BlockSpec((1, P, 128), lambda b: (b, 0, 0)),
            pl.BlockSpec((1, P, 128), lambda b: (b, 0, 0)),
            pl.BlockSpec((1, P, K), lambda b: (b, 0, 0)),
        ],
        out_shape=[
            jax.ShapeDtypeStruct((B, P, 64), _f32),
            jax.ShapeDtypeStruct((B, P, 128), _f32),
            jax.ShapeDtypeStruct((B, P, 128), _f32),
            jax.ShapeDtypeStruct((B, P, K), jnp.int32),
        ],
    )(xt, wu1, wv1, col(b1a), W1b.T, col(b1b), W1c.T, col(b1c),
      w2u, w2v, row(b2))

    m = _get_sc_gather_max()(v2r.reshape(N, 128),
                             idx2.reshape(N * K))

    pooled = pl.pallas_call(
        _stage2_kernel,
        grid=(B,),
        in_specs=[
            pl.BlockSpec((1, P, 64), lambda b: (b, 0, 0)),
            pl.BlockSpec((1, P, 128), lambda b: (b, 0, 0)),
            pl.BlockSpec((P, 128), lambda b: (b, 0)),
            _full((192, 1024)), _full((1, 1024)),
        ],
        out_specs=pl.BlockSpec((1, 1, 1024), lambda b: (b, 0, 0)),
        out_shape=jax.ShapeDtypeStruct((B, 1, 1024), _f32),
    )(x1r, u2r, m, Wl, row(bl))

    return pl.pallas_call(
        _head_kernel,
        in_specs=[_full((B, 1024)),
                  _full((1024, 512)), _full((1, 512)),
                  _full((512, 256)), _full((1, 256)),
                  _full((256, 16)), _full((1, 16))],
        out_specs=_full((B, 16)),
        out_shape=jax.ShapeDtypeStruct((B, 16), _f32),
    )(pooled.reshape(B, 1024), Wm1, row(bm1), Wm2, row(bm2), Wm3, row(bm3))
